# baseline (speedup 1.0000x reference)
import jax
import jax.numpy as jnp
from jax.experimental import pallas as pl
from jax.experimental.pallas import tpu as pltpu

_N = 1024
_D = 512
_TOPK = 10
_HALF = 5
_SIGMA = 1.0
_DELTA = 1.0


def _self_d2(x):
    xb = x.astype(jnp.bfloat16)
    g = jax.lax.dot_general(
        xb, xb, (((1,), (1,)), ((), ())), preferred_element_type=jnp.float32
    )
    return jnp.maximum(2.0 - 2.0 * g, 0.0)


def _stml_kernel(s_ref, t_ref, idxc_ref, idxr_ref, out_ref):
    n = _N
    s = s_ref[...]
    t = t_ref[...]
    s = s / jnp.maximum(
        jnp.sqrt(jnp.sum(s * s, axis=1, keepdims=True)), 1e-12
    )
    t = t / jnp.maximum(
        jnp.sqrt(jnp.sum(t * t, axis=1, keepdims=True)), 1e-12
    )

    d2s = _self_d2(s)
    s_dist = jnp.sqrt(d2s)
    mean_t = jnp.mean(s_dist, axis=0, keepdims=True)
    s_dist = s_dist * (1.0 / mean_t).T

    d2t = _self_d2(t)
    tiny = d2t <= 1e-12
    w_p = jnp.exp(-d2t / _SIGMA)

    same = idxc_ref[...] == idxr_ref[...]

    rowi = jax.lax.broadcasted_iota(jnp.int32, (n, n), 0)
    d2bits = jax.lax.bitcast_convert_type(d2t, jnp.int32)
    keyT = jnp.where(same | tiny, 0, d2bits & ~jnp.int32(1023)) | rowi

    big = jnp.int32(2147483647)
    w_half_t = None
    for k in range(_TOPK):
        colmin = jnp.min(keyT, axis=0, keepdims=True)
        keyT = jnp.where(keyT == colmin, big, keyT)
        if k == _HALF - 1:
            w_half_t = (keyT == big).astype(jnp.float32)
    w_nn_t = (keyT == big).astype(jnp.float32)

    v = w_nn_t.T * w_nn_t
    cnt = jnp.sum(v, axis=0)
    v_bf = v.astype(jnp.bfloat16)
    m = jax.lax.dot_general(
        v_bf, v_bf, (((1,), (0,)), ((), ())),
        preferred_element_type=jnp.float32,
    )
    rc = 0.1 / jnp.maximum(cnt, 1.0)
    x_half = jax.lax.dot_general(
        w_half_t.astype(jnp.bfloat16),
        (v * m * rc[:, None]).astype(jnp.bfloat16),
        (((0,), (0,)), ((), ())),
        preferred_element_type=jnp.float32,
    )

    rp = jnp.maximum(_DELTA - s_dist, 0.0)
    rp2 = rp * rp
    a2 = 0.5 * (s_dist * s_dist - rp2)
    f = rp2 + a2 * w_p + (a2 + a2.T) * x_half
    col = jax.lax.broadcasted_iota(jnp.int32, (n, n), 1)
    loss = jnp.sum(jnp.where(rowi == col, 0.0, f)) / float(n * (n - 1))
    out_ref[...] = jnp.reshape(loss, (1, 1))


def kernel(s_emb, t_emb, idx):
    idx_col = idx.reshape(_N, 1)
    idx_row = idx.reshape(1, _N)
    out = pl.pallas_call(
        _stml_kernel,
        out_shape=jax.ShapeDtypeStruct((1, 1), jnp.float32),
    )(s_emb, t_emb, idx_col, idx_row)
    return out[0, 0]


# speedup vs baseline: 1.0148x; 1.0148x over previous
import jax
import jax.numpy as jnp
from jax.experimental import pallas as pl
from jax.experimental.pallas import tpu as pltpu

_N = 1024
_D = 512
_TOPK = 10
_HALF = 5
_SIGMA = 1.0
_DELTA = 1.0


def _self_d2(x):
    xb = x.astype(jnp.bfloat16)
    g = jax.lax.dot_general(
        xb, xb, (((1,), (1,)), ((), ())), preferred_element_type=jnp.float32
    )
    return jnp.maximum(2.0 - 2.0 * g, 0.0)


def _stml_kernel(s_ref, t_ref, idxc_ref, idxr_ref, out_ref):
    n = _N
    s = s_ref[...]
    t = t_ref[...]
    s = s / jnp.maximum(
        jnp.sqrt(jnp.sum(s * s, axis=1, keepdims=True)), 1e-12
    )
    t = t / jnp.maximum(
        jnp.sqrt(jnp.sum(t * t, axis=1, keepdims=True)), 1e-12
    )

    d2s = _self_d2(s)
    s_dist = jnp.sqrt(d2s)
    s_dist = s_dist / jnp.mean(s_dist, axis=1, keepdims=True)

    d2t = _self_d2(t)
    tiny = d2t <= 1e-12
    w_p = jnp.exp(-d2t / _SIGMA)

    same = idxc_ref[...] == idxr_ref[...]

    rowi = jax.lax.broadcasted_iota(jnp.int32, (n, n), 0)
    d2bits = jax.lax.bitcast_convert_type(d2t, jnp.int32)
    keyT = jnp.where(same | tiny, 0, d2bits & ~jnp.int32(1023)) | rowi

    big = jnp.int32(2147483647)
    w_half_t = None
    for k in range(_TOPK):
        colmin = jnp.min(keyT, axis=0, keepdims=True)
        keyT = jnp.where(keyT == colmin, big, keyT)
        if k == _HALF - 1:
            w_half_t = (keyT == big).astype(jnp.float32)
    w_nn_t = (keyT == big).astype(jnp.float32)

    v = w_nn_t.T * w_nn_t
    cnt = jnp.sum(v, axis=0)
    v_bf = v.astype(jnp.bfloat16)
    m = jax.lax.dot_general(
        v_bf, v_bf, (((1,), (0,)), ((), ())),
        preferred_element_type=jnp.float32,
    )
    rc = 0.1 / jnp.maximum(cnt, 1.0)
    x_half = jax.lax.dot_general(
        w_half_t.astype(jnp.bfloat16),
        (v * m * rc[:, None]).astype(jnp.bfloat16),
        (((0,), (0,)), ((), ())),
        preferred_element_type=jnp.float32,
    )

    rp = jnp.maximum(_DELTA - s_dist, 0.0)
    rp2 = rp * rp
    a2 = 0.5 * (s_dist * s_dist - rp2)
    f = rp2 + a2 * w_p + (a2 + a2.T) * x_half
    col = jax.lax.broadcasted_iota(jnp.int32, (n, n), 1)
    loss = jnp.sum(jnp.where(rowi == col, 0.0, f)) / float(n * (n - 1))
    out_ref[...] = jnp.reshape(loss, (1, 1))


def kernel(s_emb, t_emb, idx):
    idx_col = idx.reshape(_N, 1)
    idx_row = idx.reshape(1, _N)
    out = pl.pallas_call(
        _stml_kernel,
        out_shape=jax.ShapeDtypeStruct((1, 1), jnp.float32),
    )(s_emb, t_emb, idx_col, idx_row)
    return out[0, 0]


# speedup vs baseline: 1.0159x; 1.0011x over previous
import jax
import jax.numpy as jnp
from jax.experimental import pallas as pl

_N = 1024
_D = 512
_TOPK = 10
_HALF = 5
_SIGMA = 1.0
_DELTA = 1.0


def _self_d2(x):
    xb = x.astype(jnp.bfloat16)
    g = jax.lax.dot_general(
        xb, xb, (((1,), (1,)), ((), ())), preferred_element_type=jnp.float32
    )
    return jnp.maximum(2.0 - 2.0 * g, 0.0)


def _stml_kernel(s_ref, t_ref, idxc_ref, idxr_ref, out_ref):
    n = _N
    s = s_ref[...]
    t = t_ref[...]
    s = s / jnp.maximum(
        jnp.sqrt(jnp.sum(s * s, axis=1, keepdims=True)), 1e-12
    )
    t = t / jnp.maximum(
        jnp.sqrt(jnp.sum(t * t, axis=1, keepdims=True)), 1e-12
    )

    d2s = _self_d2(s)
    s_dist = jnp.sqrt(d2s)
    s_dist = s_dist / jnp.mean(s_dist, axis=1, keepdims=True)

    d2t = _self_d2(t)
    tiny = d2t <= 1e-12
    w_p = jnp.exp(-d2t / _SIGMA)

    same = idxc_ref[...] == idxr_ref[...]

    rowi = jax.lax.broadcasted_iota(jnp.int32, (n, n), 0)
    d2bits = jax.lax.bitcast_convert_type(d2t, jnp.int32)
    keyT = jnp.where(same | tiny, 0, d2bits & ~jnp.int32(1023)) | rowi

    big = jnp.int32(2147483647)
    w_half_t = None
    for k in range(_TOPK):
        colmin = jnp.min(keyT, axis=0, keepdims=True)
        keyT = jnp.where(keyT == colmin, big, keyT)
        if k == _HALF - 1:
            w_half_t = (keyT == big).astype(jnp.float32)
    w_nn_t = (keyT == big).astype(jnp.float32)

    v = w_nn_t.T * w_nn_t
    cnt = jnp.sum(v, axis=0)
    v_bf = v.astype(jnp.bfloat16)
    m = jax.lax.dot_general(
        v_bf, v_bf, (((1,), (0,)), ((), ())),
        preferred_element_type=jnp.float32,
    )
    rc = 0.1 / jnp.maximum(cnt, 1.0)
    x_half = jax.lax.dot_general(
        w_half_t.astype(jnp.bfloat16),
        (v * m * rc[:, None]).astype(jnp.bfloat16),
        (((0,), (0,)), ((), ())),
        preferred_element_type=jnp.float32,
    )

    rp = jnp.maximum(_DELTA - s_dist, 0.0)
    rp2 = rp * rp
    a2 = 0.5 * (s_dist * s_dist - rp2)
    f = rp2 + a2 * w_p + (a2 + a2.T) * x_half
    col = jax.lax.broadcasted_iota(jnp.int32, (n, n), 1)
    loss = jnp.sum(jnp.where(rowi == col, 0.0, f)) / float(n * (n - 1))
    out_ref[...] = jnp.reshape(loss, (1, 1))


def kernel(s_emb, t_emb, idx):
    idx_col = idx.reshape(_N, 1)
    idx_row = idx.reshape(1, _N)
    out = pl.pallas_call(
        _stml_kernel,
        out_shape=jax.ShapeDtypeStruct((1, 1), jnp.float32),
    )(s_emb, t_emb, idx_col, idx_row)
    return out[0, 0]
